# baseline (speedup 1.0000x reference)
import functools

import jax
import jax.numpy as jnp
from jax import lax
from jax.experimental import pallas as pl
from jax.experimental.pallas import tpu as pltpu
from jax.experimental.pallas import tpu_sc as plsc

N = 10000
HALF = 5000
E = 320000
F = 16
NSLICE = 16
E_PAD = 327680
SL = E_PAD // NSLICE
C = 1024
NCH = SL // C
SUB = 128

_mesh = plsc.VectorSubcoreMesh(core_axis_name="c", subcore_axis_name="s")


@functools.partial(
    pl.kernel,
    out_type=jax.ShapeDtypeStruct((NSLICE, N, F), jnp.float32),
    mesh=_mesh,
    compiler_params=pltpu.CompilerParams(needs_layout_passes=False,
                                         use_tc_tiling_on_sc=False),
    scratch_types=[
        pltpu.VMEM((C,), jnp.int32),
        pltpu.VMEM((C,), jnp.float32),
        pltpu.VMEM((HALF, F), jnp.float32),
    ],
)
def _deg_kernel(col_hbm, ew_hbm, out_hbm, col_v, ew_v, acc_v):
    h = lax.axis_index("c")
    s = lax.axis_index("s")
    zero16 = jnp.zeros((F,), jnp.float32)

    def zb(i, carry):
        base = i * 20
        for t in range(20):
            acc_v[base + t] = zero16
        return carry

    lax.fori_loop(0, HALF // 20, zb, 0)

    lane = lax.iota(jnp.int32, 16)
    base = s * SL
    for k in range(NCH):
        off = base + k * C
        pltpu.sync_copy(col_hbm.at[pl.ds(off, C)], col_v)
        pltpu.sync_copy(ew_hbm.at[pl.ds(off, C)], ew_v)

        def db(i, carry):
            sl = pl.ds(i * 16, 16)
            c16 = col_v[sl]
            e16 = ew_v[sl]
            cl = c16 - h * HALF
            ok = (cl >= 0) & (cl < HALF)
            r = jnp.where(ok, cl, 0)
            w = jnp.where(ok, e16, 0.0)
            plsc.addupdate_scatter(acc_v, [r, lane], w)
            return carry

        lax.fori_loop(0, C // 16, db, 0, unroll=4)

    pltpu.sync_copy(acc_v, out_hbm.at[s, pl.ds(h * HALF, HALF)])


@functools.partial(
    pl.kernel,
    out_type=jax.ShapeDtypeStruct((NSLICE, N, F), jnp.float32),
    mesh=_mesh,
    compiler_params=pltpu.CompilerParams(needs_layout_passes=False,
                                         use_tc_tiling_on_sc=False),
    scratch_types=[
        pltpu.VMEM((N,), jnp.float32),
        pltpu.VMEM((C,), jnp.int32),
        pltpu.VMEM((C,), jnp.int32),
        pltpu.VMEM((C,), jnp.float32),
        pltpu.VMEM((C,), jnp.float32),
        pltpu.VMEM((C,), jnp.int32),
        pltpu.VMEM((C, F), jnp.float32),
        pltpu.VMEM((HALF, F), jnp.float32),
        pltpu.SemaphoreType.DMA,
    ],
)
def _agg_kernel(row_hbm, col_hbm, ew_hbm, dinv_hbm, table_hbm, out_hbm,
                dinv_v, row_v, col_v, ew_v, norm_v, addr_v, rows_v, acc_v,
                gsem):
    h = lax.axis_index("c")
    s = lax.axis_index("s")
    pltpu.sync_copy(dinv_hbm, dinv_v)

    zero16 = jnp.zeros((F,), jnp.float32)

    def zb(i, carry):
        base = i * 20
        for t in range(20):
            acc_v[base + t] = zero16
        return carry

    lax.fori_loop(0, HALF // 20, zb, 0)

    lane = lax.iota(jnp.int32, 16)
    base = s * SL
    for k in range(NCH):
        off = base + k * C
        pltpu.sync_copy(row_hbm.at[pl.ds(off, C)], row_v)
        pltpu.sync_copy(col_hbm.at[pl.ds(off, C)], col_v)
        pltpu.sync_copy(ew_hbm.at[pl.ds(off, C)], ew_v)

        descs = [
            pltpu.async_copy(
                table_hbm.at[row_v.at[pl.ds(j * SUB, SUB)]],
                rows_v.at[pl.ds(j * SUB, SUB)],
                gsem,
            )
            for j in range(C // SUB)
        ]

        def nb(i, carry):
            sl = pl.ds(i * 16, 16)
            r16 = row_v[sl]
            c16 = col_v[sl]
            e16 = ew_v[sl]
            dr = plsc.load_gather(dinv_v, [r16])
            dc = plsc.load_gather(dinv_v, [c16])
            cl = c16 - h * HALF
            ok = (cl >= 0) & (cl < HALF)
            norm_v[sl] = jnp.where(ok, dr * e16 * dc, 0.0)
            addr_v[sl] = jnp.where(ok, cl, 0)
            return carry

        lax.fori_loop(0, C // 16, nb, 0, unroll=4)

        for d in descs:
            d.wait()

        def eb(e, carry):
            e_vec = lax.broadcast(e, (16,))
            wb = plsc.load_gather(norm_v, [e_vec])
            ab = plsc.load_gather(addr_v, [e_vec])
            plsc.addupdate_scatter(acc_v, [ab, lane], wb * rows_v[e])
            return carry

        lax.fori_loop(0, C, eb, 0, unroll=8)

    pltpu.sync_copy(acc_v, out_hbm.at[s, pl.ds(h * HALF, HALF)])


def _k2_body(parts_ref, x_ref, w1_ref, dinv_ref, xw1_ref, acc_ref):
    i = pl.program_id(0)

    @pl.when(i == 0)
    def _():
        acc_ref[...] = jnp.zeros_like(acc_ref)

    acc_ref[...] += parts_ref[0]

    @pl.when(i == NSLICE - 1)
    def _():
        deg = jnp.sum(acc_ref[...], axis=1) + 1.0
        dinv_ref[...] = jnp.where(deg > 0, lax.rsqrt(deg), 0.0)
        xw1_ref[...] = jnp.dot(x_ref[...], w1_ref[...],
                               preferred_element_type=jnp.float32)


def _k4_body(parts_ref, xw1_ref, dinv_ref, b1_ref, w2p_ref, xw2p_ref,
             acc_ref):
    i = pl.program_id(0)

    @pl.when(i == 0)
    def _():
        acc_ref[...] = jnp.zeros_like(acc_ref)

    acc_ref[...] += parts_ref[0]

    @pl.when(i == NSLICE - 1)
    def _():
        d2 = dinv_ref[...] ** 2
        h1 = acc_ref[...] + d2[:, None] * xw1_ref[...] + b1_ref[...][None, :]
        h1 = jnp.maximum(h1, 0.0)
        xw2p_ref[...] = jnp.dot(h1, w2p_ref[...],
                                preferred_element_type=jnp.float32)


def _k6_body(parts_ref, xw2p_ref, dinv_ref, b2_ref, out_ref, acc_ref):
    i = pl.program_id(0)

    @pl.when(i == 0)
    def _():
        acc_ref[...] = jnp.zeros_like(acc_ref)

    acc_ref[...] += parts_ref[0]

    @pl.when(i == NSLICE - 1)
    def _():
        d2 = dinv_ref[...] ** 2
        o = (acc_ref[...][:, :2] + d2[:, None] * xw2p_ref[...][:, :2]
             + b2_ref[...][None, :])
        out_ref[...] = jax.nn.log_softmax(o, axis=1)


_full = lambda *block: pl.BlockSpec(block, lambda i: tuple(0 for _ in block))

_k2_call = pl.pallas_call(
    _k2_body,
    grid=(NSLICE,),
    in_specs=[
        pl.BlockSpec((1, N, F), lambda i: (i, 0, 0)),
        _full(N, 128),
        _full(128, F),
    ],
    out_specs=(_full(N), _full(N, F)),
    scratch_shapes=[pltpu.VMEM((N, F), jnp.float32)],
    out_shape=(jax.ShapeDtypeStruct((N,), jnp.float32),
               jax.ShapeDtypeStruct((N, F), jnp.float32)),
)

_k4_call = pl.pallas_call(
    _k4_body,
    grid=(NSLICE,),
    in_specs=[
        pl.BlockSpec((1, N, F), lambda i: (i, 0, 0)),
        _full(N, F),
        _full(N),
        _full(F),
        _full(F, F),
    ],
    out_specs=_full(N, F),
    scratch_shapes=[pltpu.VMEM((N, F), jnp.float32)],
    out_shape=jax.ShapeDtypeStruct((N, F), jnp.float32),
)

_k6_call = pl.pallas_call(
    _k6_body,
    grid=(NSLICE,),
    in_specs=[
        pl.BlockSpec((1, N, F), lambda i: (i, 0, 0)),
        _full(N, F),
        _full(N),
        _full(2),
    ],
    out_specs=_full(N, 2),
    scratch_shapes=[pltpu.VMEM((N, F), jnp.float32)],
    out_shape=jax.ShapeDtypeStruct((N, 2), jnp.float32),
)


def kernel(x, edge_index, edge_weight, W1, b1, W2, b2):
    row = edge_index[0]
    col = edge_index[1]
    pad = E_PAD - row.shape[0]
    zi = jnp.zeros((pad,), row.dtype)
    row_p = jnp.concatenate([row, zi])
    col_p = jnp.concatenate([col, zi])
    ew_p = jnp.concatenate([edge_weight, jnp.zeros((pad,), edge_weight.dtype)])
    w2p = jnp.zeros((F, F), W2.dtype).at[:, :2].set(W2)

    deg_parts = _deg_kernel(col_p, ew_p)
    dinv, xw1 = _k2_call(deg_parts, x, W1)
    parts1 = _agg_kernel(row_p, col_p, ew_p, dinv, xw1)
    xw2p = _k4_call(parts1, xw1, dinv, b1, w2p)
    parts2 = _agg_kernel(row_p, col_p, ew_p, dinv, xw2p)
    return _k6_call(parts2, xw2p, dinv, b2)


# speedup vs baseline: 1.3863x; 1.3863x over previous
import functools

import jax
import jax.numpy as jnp
from jax import lax
from jax.experimental import pallas as pl
from jax.experimental.pallas import tpu as pltpu
from jax.experimental.pallas import tpu_sc as plsc

N = 10000
HALF = 5000
E = 320000
F = 16
NSLICE = 16
E_PAD = 327680
SL = E_PAD // NSLICE
C = 1024
NCH = SL // C
SUB = 128

_mesh = plsc.VectorSubcoreMesh(core_axis_name="c", subcore_axis_name="s")


@functools.partial(
    pl.kernel,
    out_type=jax.ShapeDtypeStruct((NSLICE, N, F), jnp.float32),
    mesh=_mesh,
    compiler_params=pltpu.CompilerParams(needs_layout_passes=False,
                                         use_tc_tiling_on_sc=False),
    scratch_types=[
        pltpu.VMEM((C,), jnp.int32),
        pltpu.VMEM((C,), jnp.float32),
        pltpu.VMEM((HALF, F), jnp.float32),
    ],
)
def _deg_kernel(col_hbm, ew_hbm, out_hbm, col_v, ew_v, acc_v):
    h = lax.axis_index("c")
    s = lax.axis_index("s")
    zero16 = jnp.zeros((F,), jnp.float32)

    def zb(i, carry):
        base = i * 20
        for t in range(20):
            acc_v[base + t] = zero16
        return carry

    lax.fori_loop(0, HALF // 20, zb, 0)

    lane = lax.iota(jnp.int32, 16)
    base = s * SL

    def chunk(k, carry):
        off = base + k * C
        pltpu.sync_copy(col_hbm.at[pl.ds(off, C)], col_v)
        pltpu.sync_copy(ew_hbm.at[pl.ds(off, C)], ew_v)

        def db(i, carry):
            sl = pl.ds(i * 16, 16)
            c16 = col_v[sl]
            e16 = ew_v[sl]
            cl = c16 - h * HALF
            ok = (cl >= 0) & (cl < HALF)
            r = jnp.where(ok, cl, 0)
            w = jnp.where(ok, e16, 0.0)
            plsc.addupdate_scatter(acc_v, [r, lane], w)
            return carry

        lax.fori_loop(0, C // 16, db, 0, unroll=4)
        return carry

    lax.fori_loop(0, NCH, chunk, 0)

    pltpu.sync_copy(acc_v, out_hbm.at[s, pl.ds(h * HALF, HALF)])


@functools.partial(
    pl.kernel,
    out_type=jax.ShapeDtypeStruct((NSLICE, N, F), jnp.float32),
    mesh=_mesh,
    compiler_params=pltpu.CompilerParams(needs_layout_passes=False,
                                         use_tc_tiling_on_sc=False),
    scratch_types=[
        pltpu.VMEM((N,), jnp.float32),
        pltpu.VMEM((C,), jnp.int32),
        pltpu.VMEM((C,), jnp.int32),
        pltpu.VMEM((C,), jnp.float32),
        pltpu.VMEM((C,), jnp.float32),
        pltpu.VMEM((C,), jnp.int32),
        pltpu.VMEM((C, F), jnp.float32),
        pltpu.VMEM((HALF, F), jnp.float32),
        pltpu.SemaphoreType.DMA,
    ],
)
def _agg_kernel(row_hbm, col_hbm, ew_hbm, dinv_hbm, table_hbm, out_hbm,
                dinv_v, row_v, col_v, ew_v, norm_v, addr_v, rows_v, acc_v,
                gsem):
    h = lax.axis_index("c")
    s = lax.axis_index("s")
    pltpu.sync_copy(dinv_hbm, dinv_v)

    zero16 = jnp.zeros((F,), jnp.float32)

    def zb(i, carry):
        base = i * 20
        for t in range(20):
            acc_v[base + t] = zero16
        return carry

    lax.fori_loop(0, HALF // 20, zb, 0)

    lane = lax.iota(jnp.int32, 16)
    base = s * SL

    def chunk(k, carry):
        off = base + k * C
        pltpu.sync_copy(row_hbm.at[pl.ds(off, C)], row_v)
        pltpu.sync_copy(col_hbm.at[pl.ds(off, C)], col_v)
        pltpu.sync_copy(ew_hbm.at[pl.ds(off, C)], ew_v)

        descs = [
            pltpu.async_copy(
                table_hbm.at[row_v.at[pl.ds(j * SUB, SUB)]],
                rows_v.at[pl.ds(j * SUB, SUB)],
                gsem,
            )
            for j in range(C // SUB)
        ]

        def nb(i, carry):
            sl = pl.ds(i * 16, 16)
            r16 = row_v[sl]
            c16 = col_v[sl]
            e16 = ew_v[sl]
            dr = plsc.load_gather(dinv_v, [r16])
            dc = plsc.load_gather(dinv_v, [c16])
            cl = c16 - h * HALF
            ok = (cl >= 0) & (cl < HALF)
            norm_v[sl] = jnp.where(ok, dr * e16 * dc, 0.0)
            addr_v[sl] = jnp.where(ok, cl, 0)
            return carry

        lax.fori_loop(0, C // 16, nb, 0, unroll=4)

        for d in descs:
            d.wait()

        @plsc.parallel_loop(0, C // 16, 1, unroll=2)
        def eb(i):
            sl = pl.ds(i * 16, 16)
            a16 = addr_v[sl]
            w16 = norm_v[sl]
            for t in range(16):
                plsc.addupdate(acc_v.at[a16[t]], w16[t] * rows_v[i * 16 + t])

        return carry

    lax.fori_loop(0, NCH, chunk, 0)

    pltpu.sync_copy(acc_v, out_hbm.at[s, pl.ds(h * HALF, HALF)])


def _k2_body(parts_ref, x_ref, w1_ref, dinv_ref, xw1_ref, acc_ref):
    i = pl.program_id(0)

    @pl.when(i == 0)
    def _():
        acc_ref[...] = jnp.zeros_like(acc_ref)

    acc_ref[...] += parts_ref[0]

    @pl.when(i == NSLICE - 1)
    def _():
        deg = jnp.sum(acc_ref[...], axis=1) + 1.0
        dinv_ref[...] = jnp.where(deg > 0, lax.rsqrt(deg), 0.0)
        xw1_ref[...] = jnp.dot(x_ref[...], w1_ref[...],
                               preferred_element_type=jnp.float32)


def _k4_body(parts_ref, xw1_ref, dinv_ref, b1_ref, w2p_ref, xw2p_ref,
             acc_ref):
    i = pl.program_id(0)

    @pl.when(i == 0)
    def _():
        acc_ref[...] = jnp.zeros_like(acc_ref)

    acc_ref[...] += parts_ref[0]

    @pl.when(i == NSLICE - 1)
    def _():
        d2 = dinv_ref[...] ** 2
        h1 = acc_ref[...] + d2[:, None] * xw1_ref[...] + b1_ref[...][None, :]
        h1 = jnp.maximum(h1, 0.0)
        xw2p_ref[...] = jnp.dot(h1, w2p_ref[...],
                                preferred_element_type=jnp.float32)


def _k6_body(parts_ref, xw2p_ref, dinv_ref, b2_ref, out_ref, acc_ref):
    i = pl.program_id(0)

    @pl.when(i == 0)
    def _():
        acc_ref[...] = jnp.zeros_like(acc_ref)

    acc_ref[...] += parts_ref[0]

    @pl.when(i == NSLICE - 1)
    def _():
        d2 = dinv_ref[...] ** 2
        o = (acc_ref[...][:, :2] + d2[:, None] * xw2p_ref[...][:, :2]
             + b2_ref[...][None, :])
        out_ref[...] = jax.nn.log_softmax(o, axis=1)


_full = lambda *block: pl.BlockSpec(block, lambda i: tuple(0 for _ in block))

_k2_call = pl.pallas_call(
    _k2_body,
    grid=(NSLICE,),
    in_specs=[
        pl.BlockSpec((1, N, F), lambda i: (i, 0, 0)),
        _full(N, 128),
        _full(128, F),
    ],
    out_specs=(_full(N), _full(N, F)),
    scratch_shapes=[pltpu.VMEM((N, F), jnp.float32)],
    out_shape=(jax.ShapeDtypeStruct((N,), jnp.float32),
               jax.ShapeDtypeStruct((N, F), jnp.float32)),
)

_k4_call = pl.pallas_call(
    _k4_body,
    grid=(NSLICE,),
    in_specs=[
        pl.BlockSpec((1, N, F), lambda i: (i, 0, 0)),
        _full(N, F),
        _full(N),
        _full(F),
        _full(F, F),
    ],
    out_specs=_full(N, F),
    scratch_shapes=[pltpu.VMEM((N, F), jnp.float32)],
    out_shape=jax.ShapeDtypeStruct((N, F), jnp.float32),
)

_k6_call = pl.pallas_call(
    _k6_body,
    grid=(NSLICE,),
    in_specs=[
        pl.BlockSpec((1, N, F), lambda i: (i, 0, 0)),
        _full(N, F),
        _full(N),
        _full(2),
    ],
    out_specs=_full(N, 2),
    scratch_shapes=[pltpu.VMEM((N, F), jnp.float32)],
    out_shape=jax.ShapeDtypeStruct((N, 2), jnp.float32),
)


def kernel(x, edge_index, edge_weight, W1, b1, W2, b2):
    row = edge_index[0]
    col = edge_index[1]
    pad = E_PAD - row.shape[0]
    zi = jnp.zeros((pad,), row.dtype)
    row_p = jnp.concatenate([row, zi])
    col_p = jnp.concatenate([col, zi])
    ew_p = jnp.concatenate([edge_weight, jnp.zeros((pad,), edge_weight.dtype)])
    w2p = jnp.zeros((F, F), W2.dtype).at[:, :2].set(W2)

    deg_parts = _deg_kernel(col_p, ew_p)
    dinv, xw1 = _k2_call(deg_parts, x, W1)
    parts1 = _agg_kernel(row_p, col_p, ew_p, dinv, xw1)
    xw2p = _k4_call(parts1, xw1, dinv, b1, w2p)
    parts2 = _agg_kernel(row_p, col_p, ew_p, dinv, xw2p)
    return _k6_call(parts2, xw2p, dinv, b2)


# speedup vs baseline: 1.4003x; 1.0101x over previous
import functools

import jax
import jax.numpy as jnp
from jax import lax
from jax.experimental import pallas as pl
from jax.experimental.pallas import tpu as pltpu
from jax.experimental.pallas import tpu_sc as plsc

N = 10000
HALF = 5000
E = 320000
F = 16
NSLICE = 16
E_PAD = 327680
SL = E_PAD // NSLICE
C = 1024
NCH = SL // C
SUB = 128

_mesh = plsc.VectorSubcoreMesh(core_axis_name="c", subcore_axis_name="s")


@functools.partial(
    pl.kernel,
    out_type=jax.ShapeDtypeStruct((NSLICE, N, F), jnp.float32),
    mesh=_mesh,
    compiler_params=pltpu.CompilerParams(needs_layout_passes=False,
                                         use_tc_tiling_on_sc=False),
    scratch_types=[
        pltpu.VMEM((C,), jnp.int32),
        pltpu.VMEM((C,), jnp.float32),
        pltpu.VMEM((HALF, F), jnp.float32),
    ],
)
def _deg_kernel(col_hbm, ew_hbm, out_hbm, col_v, ew_v, acc_v):
    h = lax.axis_index("c")
    s = lax.axis_index("s")
    zero16 = jnp.zeros((F,), jnp.float32)

    def zb(i, carry):
        base = i * 20
        for t in range(20):
            acc_v[base + t] = zero16
        return carry

    lax.fori_loop(0, HALF // 20, zb, 0)

    lane = lax.iota(jnp.int32, 16)
    base = s * SL

    def chunk(k, carry):
        off = base + k * C
        pltpu.sync_copy(col_hbm.at[pl.ds(off, C)], col_v)
        pltpu.sync_copy(ew_hbm.at[pl.ds(off, C)], ew_v)

        @plsc.parallel_loop(0, C // 16, 1, unroll=4)
        def db(i):
            sl = pl.ds(i * 16, 16)
            c16 = col_v[sl]
            e16 = ew_v[sl]
            cl = c16 - h * HALF
            ok = (cl >= 0) & (cl < HALF)
            r = jnp.where(ok, cl, 0)
            w = jnp.where(ok, e16, 0.0)
            plsc.addupdate_scatter(acc_v, [r, lane], w)
        return carry

    lax.fori_loop(0, NCH, chunk, 0)

    pltpu.sync_copy(acc_v, out_hbm.at[s, pl.ds(h * HALF, HALF)])


@functools.partial(
    pl.kernel,
    out_type=jax.ShapeDtypeStruct((NSLICE, N, F), jnp.float32),
    mesh=_mesh,
    compiler_params=pltpu.CompilerParams(needs_layout_passes=False,
                                         use_tc_tiling_on_sc=False),
    scratch_types=[
        pltpu.VMEM((N,), jnp.float32),
        pltpu.VMEM((C,), jnp.int32),
        pltpu.VMEM((C,), jnp.int32),
        pltpu.VMEM((C,), jnp.float32),
        pltpu.VMEM((C,), jnp.float32),
        pltpu.VMEM((C,), jnp.int32),
        pltpu.VMEM((C, F), jnp.float32),
        pltpu.VMEM((HALF, F), jnp.float32),
        pltpu.SemaphoreType.DMA,
    ],
)
def _agg_kernel(row_hbm, col_hbm, ew_hbm, dinv_hbm, table_hbm, out_hbm,
                dinv_v, row_v, col_v, ew_v, norm_v, addr_v, rows_v, acc_v,
                gsem):
    h = lax.axis_index("c")
    s = lax.axis_index("s")
    pltpu.sync_copy(dinv_hbm, dinv_v)

    zero16 = jnp.zeros((F,), jnp.float32)

    def zb(i, carry):
        base = i * 20
        for t in range(20):
            acc_v[base + t] = zero16
        return carry

    lax.fori_loop(0, HALF // 20, zb, 0)

    lane = lax.iota(jnp.int32, 16)
    base = s * SL

    def chunk(k, carry):
        off = base + k * C
        pltpu.sync_copy(row_hbm.at[pl.ds(off, C)], row_v)
        pltpu.sync_copy(col_hbm.at[pl.ds(off, C)], col_v)
        pltpu.sync_copy(ew_hbm.at[pl.ds(off, C)], ew_v)

        descs = [
            pltpu.async_copy(
                table_hbm.at[row_v.at[pl.ds(j * SUB, SUB)]],
                rows_v.at[pl.ds(j * SUB, SUB)],
                gsem,
            )
            for j in range(C // SUB)
        ]

        def nb(i):
            sl = pl.ds(i * 16, 16)
            r16 = row_v[sl]
            c16 = col_v[sl]
            e16 = ew_v[sl]
            dr = plsc.load_gather(dinv_v, [r16])
            dc = plsc.load_gather(dinv_v, [c16])
            cl = c16 - h * HALF
            ok = (cl >= 0) & (cl < HALF)
            norm_v[sl] = jnp.where(ok, dr * e16 * dc, 0.0)
            addr_v[sl] = jnp.where(ok, cl, 0)

        nb_loop = plsc.parallel_loop(0, C // 16, 1, unroll=4)(nb)

        for d in descs:
            d.wait()

        @plsc.parallel_loop(0, C // 16, 1, unroll=4)
        def eb(i):
            sl = pl.ds(i * 16, 16)
            a16 = addr_v[sl]
            w16 = norm_v[sl]
            for t in range(16):
                plsc.addupdate(acc_v.at[a16[t]], w16[t] * rows_v[i * 16 + t])

        return carry

    lax.fori_loop(0, NCH, chunk, 0)

    pltpu.sync_copy(acc_v, out_hbm.at[s, pl.ds(h * HALF, HALF)])


def _k2_body(parts_ref, x_ref, w1_ref, dinv_ref, xw1_ref, acc_ref):
    i = pl.program_id(0)

    @pl.when(i == 0)
    def _():
        acc_ref[...] = jnp.zeros_like(acc_ref)

    acc_ref[...] += parts_ref[0]

    @pl.when(i == NSLICE - 1)
    def _():
        deg = jnp.sum(acc_ref[...], axis=1) + 1.0
        dinv_ref[...] = jnp.where(deg > 0, lax.rsqrt(deg), 0.0)
        xw1_ref[...] = jnp.dot(x_ref[...], w1_ref[...],
                               preferred_element_type=jnp.float32)


def _k4_body(parts_ref, xw1_ref, dinv_ref, b1_ref, w2p_ref, xw2p_ref,
             acc_ref):
    i = pl.program_id(0)

    @pl.when(i == 0)
    def _():
        acc_ref[...] = jnp.zeros_like(acc_ref)

    acc_ref[...] += parts_ref[0]

    @pl.when(i == NSLICE - 1)
    def _():
        d2 = dinv_ref[...] ** 2
        h1 = acc_ref[...] + d2[:, None] * xw1_ref[...] + b1_ref[...][None, :]
        h1 = jnp.maximum(h1, 0.0)
        xw2p_ref[...] = jnp.dot(h1, w2p_ref[...],
                                preferred_element_type=jnp.float32)


def _k6_body(parts_ref, xw2p_ref, dinv_ref, b2_ref, out_ref, acc_ref):
    i = pl.program_id(0)

    @pl.when(i == 0)
    def _():
        acc_ref[...] = jnp.zeros_like(acc_ref)

    acc_ref[...] += parts_ref[0]

    @pl.when(i == NSLICE - 1)
    def _():
        d2 = dinv_ref[...] ** 2
        o = (acc_ref[...][:, :2] + d2[:, None] * xw2p_ref[...][:, :2]
             + b2_ref[...][None, :])
        out_ref[...] = jax.nn.log_softmax(o, axis=1)


_full = lambda *block: pl.BlockSpec(block, lambda i: tuple(0 for _ in block))

_k2_call = pl.pallas_call(
    _k2_body,
    grid=(NSLICE,),
    in_specs=[
        pl.BlockSpec((1, N, F), lambda i: (i, 0, 0)),
        _full(N, 128),
        _full(128, F),
    ],
    out_specs=(_full(N), _full(N, F)),
    scratch_shapes=[pltpu.VMEM((N, F), jnp.float32)],
    out_shape=(jax.ShapeDtypeStruct((N,), jnp.float32),
               jax.ShapeDtypeStruct((N, F), jnp.float32)),
)

_k4_call = pl.pallas_call(
    _k4_body,
    grid=(NSLICE,),
    in_specs=[
        pl.BlockSpec((1, N, F), lambda i: (i, 0, 0)),
        _full(N, F),
        _full(N),
        _full(F),
        _full(F, F),
    ],
    out_specs=_full(N, F),
    scratch_shapes=[pltpu.VMEM((N, F), jnp.float32)],
    out_shape=jax.ShapeDtypeStruct((N, F), jnp.float32),
)

_k6_call = pl.pallas_call(
    _k6_body,
    grid=(NSLICE,),
    in_specs=[
        pl.BlockSpec((1, N, F), lambda i: (i, 0, 0)),
        _full(N, F),
        _full(N),
        _full(2),
    ],
    out_specs=_full(N, 2),
    scratch_shapes=[pltpu.VMEM((N, F), jnp.float32)],
    out_shape=jax.ShapeDtypeStruct((N, 2), jnp.float32),
)


def kernel(x, edge_index, edge_weight, W1, b1, W2, b2):
    row = edge_index[0]
    col = edge_index[1]
    pad = E_PAD - row.shape[0]
    zi = jnp.zeros((pad,), row.dtype)
    row_p = jnp.concatenate([row, zi])
    col_p = jnp.concatenate([col, zi])
    ew_p = jnp.concatenate([edge_weight, jnp.zeros((pad,), edge_weight.dtype)])
    w2p = jnp.zeros((F, F), W2.dtype).at[:, :2].set(W2)

    deg_parts = _deg_kernel(col_p, ew_p)
    dinv, xw1 = _k2_call(deg_parts, x, W1)
    parts1 = _agg_kernel(row_p, col_p, ew_p, dinv, xw1)
    xw2p = _k4_call(parts1, xw1, dinv, b1, w2p)
    parts2 = _agg_kernel(row_p, col_p, ew_p, dinv, xw2p)
    return _k6_call(parts2, xw2p, dinv, b2)


# speedup vs baseline: 1.8704x; 1.3356x over previous
import functools

import jax
import jax.numpy as jnp
from jax import lax
from jax.experimental import pallas as pl
from jax.experimental.pallas import tpu as pltpu
from jax.experimental.pallas import tpu_sc as plsc

N = 10000
HALF = 5000
E = 320000
F = 16
NSLICE = 16
E_PAD = 327680
SL = E_PAD // NSLICE
C = 1024
NCH = SL // C
SUB = 128

_mesh = plsc.VectorSubcoreMesh(core_axis_name="c", subcore_axis_name="s")


@functools.partial(
    pl.kernel,
    out_type=jax.ShapeDtypeStruct((NSLICE, N, F), jnp.float32),
    mesh=_mesh,
    compiler_params=pltpu.CompilerParams(needs_layout_passes=False,
                                         use_tc_tiling_on_sc=False),
    scratch_types=[
        pltpu.VMEM((C,), jnp.int32),
        pltpu.VMEM((C,), jnp.float32),
        pltpu.VMEM((HALF, F), jnp.float32),
    ],
)
def _deg_kernel(col_hbm, ew_hbm, out_hbm, col_v, ew_v, acc_v):
    h = lax.axis_index("c")
    s = lax.axis_index("s")
    zero16 = jnp.zeros((F,), jnp.float32)

    def zb(i, carry):
        base = i * 20
        for t in range(20):
            acc_v[base + t] = zero16
        return carry

    lax.fori_loop(0, HALF // 20, zb, 0)

    lane = lax.iota(jnp.int32, 16)
    base = s * SL

    def chunk(k, carry):
        off = base + k * C
        pltpu.sync_copy(col_hbm.at[pl.ds(off, C)], col_v)
        pltpu.sync_copy(ew_hbm.at[pl.ds(off, C)], ew_v)

        @plsc.parallel_loop(0, C // 16, 1, unroll=4)
        def db(i):
            sl = pl.ds(i * 16, 16)
            c16 = col_v[sl]
            e16 = ew_v[sl]
            cl = c16 - h * HALF
            ok = (cl >= 0) & (cl < HALF)
            r = jnp.where(ok, cl, 0)
            w = jnp.where(ok, e16, 0.0)
            plsc.addupdate_scatter(acc_v, [r, lane], w)
        return carry

    lax.fori_loop(0, NCH, chunk, 0)

    pltpu.sync_copy(acc_v, out_hbm.at[s, pl.ds(h * HALF, HALF)])


ESL = E_PAD // 32
NCH2 = ESL // C
STR = N // NSLICE


@functools.partial(
    pl.kernel,
    out_type=jax.ShapeDtypeStruct((2, N, F), jnp.float32),
    mesh=_mesh,
    compiler_params=pltpu.CompilerParams(needs_layout_passes=False,
                                         use_tc_tiling_on_sc=False),
    scratch_types=[
        pltpu.VMEM((N,), jnp.float32),
        pltpu.VMEM((C,), jnp.int32),
        pltpu.VMEM((C,), jnp.int32),
        pltpu.VMEM((C,), jnp.float32),
        pltpu.VMEM((C,), jnp.float32),
        pltpu.VMEM((C // SUB, SUB), jnp.int32),
        pltpu.VMEM((C, F), jnp.float32),
        pltpu.VMEM((C, F), jnp.float32),
        pltpu.VMEM((STR, F), jnp.float32),
        pltpu.VMEM_SHARED((N, F), jnp.float32),
        pltpu.SemaphoreType.DMA,
        pltpu.SemaphoreType.DMA,
    ],
)
def _agg_kernel(row_hbm, col_hbm, ew_hbm, dinv_hbm, table_hbm, out_hbm,
                dinv_v, row_v, col_v, ew_v, norm_v, col2_v, rows_v, scaled_v,
                stripe_v, acc_sp, gsem, ssem):
    cid = lax.axis_index("c")
    sid = lax.axis_index("s")
    pltpu.sync_copy(dinv_hbm, dinv_v)

    zero16 = jnp.zeros((F,), jnp.float32)

    def zb(i, carry):
        base = i * 25
        for t in range(25):
            stripe_v[base + t] = zero16
        return carry

    lax.fori_loop(0, STR // 25, zb, 0)
    pltpu.sync_copy(stripe_v, acc_sp.at[pl.ds(sid * STR, STR)])
    plsc.subcore_barrier()

    base = (cid * NSLICE + sid) * ESL

    def chunk(k, carry):
        off = base + k * C
        pltpu.sync_copy(row_hbm.at[pl.ds(off, C)], row_v)
        pltpu.sync_copy(col_hbm.at[pl.ds(off, C)], col_v)
        pltpu.sync_copy(ew_hbm.at[pl.ds(off, C)], ew_v)
        for j in range(C // SUB):
            pltpu.sync_copy(col_hbm.at[pl.ds(off + j * SUB, SUB)],
                            col2_v.at[j])

        descs = [
            pltpu.async_copy(
                table_hbm.at[row_v.at[pl.ds(j * SUB, SUB)]],
                rows_v.at[pl.ds(j * SUB, SUB)],
                gsem,
            )
            for j in range(C // SUB)
        ]

        def nb(i):
            sl = pl.ds(i * 16, 16)
            r16 = row_v[sl]
            c16 = col_v[sl]
            e16 = ew_v[sl]
            dr = plsc.load_gather(dinv_v, [r16])
            dc = plsc.load_gather(dinv_v, [c16])
            norm_v[sl] = dr * e16 * dc

        plsc.parallel_loop(0, C // 16, 1, unroll=4)(nb)

        for d in descs:
            d.wait()

        @plsc.parallel_loop(0, C // 16, 1, unroll=4)
        def eb(i):
            sl = pl.ds(i * 16, 16)
            w16 = norm_v[sl]
            for t in range(16):
                scaled_v[i * 16 + t] = w16[t] * rows_v[i * 16 + t]

        sdescs = [
            pltpu.async_copy(
                scaled_v.at[pl.ds(j * SUB, SUB)],
                acc_sp.at[col2_v.at[j]],
                ssem,
                add=True,
            )
            for j in range(C // SUB)
        ]
        for d in sdescs:
            d.wait()
        return carry

    lax.fori_loop(0, NCH2, chunk, 0)

    plsc.subcore_barrier()
    pltpu.sync_copy(acc_sp.at[pl.ds(sid * STR, STR)], stripe_v)
    pltpu.sync_copy(stripe_v, out_hbm.at[cid, pl.ds(sid * STR, STR)])


def _k2_body(parts_ref, x_ref, w1_ref, dinv_ref, xw1_ref, acc_ref):
    i = pl.program_id(0)

    @pl.when(i == 0)
    def _():
        acc_ref[...] = jnp.zeros_like(acc_ref)

    acc_ref[...] += parts_ref[0]

    @pl.when(i == NSLICE - 1)
    def _():
        deg = jnp.sum(acc_ref[...], axis=1) + 1.0
        dinv_ref[...] = jnp.where(deg > 0, lax.rsqrt(deg), 0.0)
        xw1_ref[...] = jnp.dot(x_ref[...], w1_ref[...],
                               preferred_element_type=jnp.float32)


def _k4_body(parts_ref, xw1_ref, dinv_ref, b1_ref, w2p_ref, xw2p_ref,
             acc_ref):
    i = pl.program_id(0)

    @pl.when(i == 0)
    def _():
        acc_ref[...] = jnp.zeros_like(acc_ref)

    acc_ref[...] += parts_ref[0]

    @pl.when(i == 1)
    def _():
        d2 = dinv_ref[...] ** 2
        h1 = acc_ref[...] + d2[:, None] * xw1_ref[...] + b1_ref[...][None, :]
        h1 = jnp.maximum(h1, 0.0)
        xw2p_ref[...] = jnp.dot(h1, w2p_ref[...],
                                preferred_element_type=jnp.float32)


def _k6_body(parts_ref, xw2p_ref, dinv_ref, b2_ref, out_ref, acc_ref):
    i = pl.program_id(0)

    @pl.when(i == 0)
    def _():
        acc_ref[...] = jnp.zeros_like(acc_ref)

    acc_ref[...] += parts_ref[0]

    @pl.when(i == 1)
    def _():
        d2 = dinv_ref[...] ** 2
        o = (acc_ref[...][:, :2] + d2[:, None] * xw2p_ref[...][:, :2]
             + b2_ref[...][None, :])
        out_ref[...] = jax.nn.log_softmax(o, axis=1)


_full = lambda *block: pl.BlockSpec(block, lambda i: tuple(0 for _ in block))

_k2_call = pl.pallas_call(
    _k2_body,
    grid=(NSLICE,),
    in_specs=[
        pl.BlockSpec((1, N, F), lambda i: (i, 0, 0)),
        _full(N, 128),
        _full(128, F),
    ],
    out_specs=(_full(N), _full(N, F)),
    scratch_shapes=[pltpu.VMEM((N, F), jnp.float32)],
    out_shape=(jax.ShapeDtypeStruct((N,), jnp.float32),
               jax.ShapeDtypeStruct((N, F), jnp.float32)),
)

_k4_call = pl.pallas_call(
    _k4_body,
    grid=(2,),
    in_specs=[
        pl.BlockSpec((1, N, F), lambda i: (i, 0, 0)),
        _full(N, F),
        _full(N),
        _full(F),
        _full(F, F),
    ],
    out_specs=_full(N, F),
    scratch_shapes=[pltpu.VMEM((N, F), jnp.float32)],
    out_shape=jax.ShapeDtypeStruct((N, F), jnp.float32),
)

_k6_call = pl.pallas_call(
    _k6_body,
    grid=(2,),
    in_specs=[
        pl.BlockSpec((1, N, F), lambda i: (i, 0, 0)),
        _full(N, F),
        _full(N),
        _full(2),
    ],
    out_specs=_full(N, 2),
    scratch_shapes=[pltpu.VMEM((N, F), jnp.float32)],
    out_shape=jax.ShapeDtypeStruct((N, 2), jnp.float32),
)


def kernel(x, edge_index, edge_weight, W1, b1, W2, b2):
    row = edge_index[0]
    col = edge_index[1]
    pad = E_PAD - row.shape[0]
    zi = jnp.zeros((pad,), row.dtype)
    row_p = jnp.concatenate([row, zi])
    col_p = jnp.concatenate([col, zi])
    ew_p = jnp.concatenate([edge_weight, jnp.zeros((pad,), edge_weight.dtype)])
    w2p = jnp.zeros((F, F), W2.dtype).at[:, :2].set(W2)

    deg_parts = _deg_kernel(col_p, ew_p)
    dinv, xw1 = _k2_call(deg_parts, x, W1)
    parts1 = _agg_kernel(row_p, col_p, ew_p, dinv, xw1)
    xw2p = _k4_call(parts1, xw1, dinv, b1, w2p)
    parts2 = _agg_kernel(row_p, col_p, ew_p, dinv, xw2p)
    return _k6_call(parts2, xw2p, dinv, b2)
